# baseline (speedup 1.0000x reference)
import jax
import jax.numpy as jnp
from jax.experimental import pallas as pl

_N = 4096
_BM = 512
_THRESHOLD = 1.0


def _spike_kernel(x_ref, w_ref, m_ref, o_ref):
    wm = (w_ref[...] * m_ref[...]).astype(jnp.bfloat16).astype(jnp.float32)
    xv = x_ref[...].astype(jnp.bfloat16).astype(jnp.float32)
    acc = jnp.sum(wm * xv[None, :], axis=1)
    o_ref[...] = (acc - _THRESHOLD >= 0.0).astype(jnp.float32)


def kernel(x, weight, weight_mask):
    grid = (_N // _BM,)
    return pl.pallas_call(
        _spike_kernel,
        grid=grid,
        in_specs=[
            pl.BlockSpec((_N,), lambda i: (0,)),
            pl.BlockSpec((_BM, _N), lambda i: (i, 0)),
            pl.BlockSpec((_BM, _N), lambda i: (i, 0)),
        ],
        out_specs=pl.BlockSpec((_BM,), lambda i: (i,)),
        out_shape=jax.ShapeDtypeStruct((_N,), jnp.float32),
    )(x, weight, weight_mask)
